# baseline (speedup 1.0000x reference)
import jax
import jax.numpy as jnp
from jax.experimental import pallas as pl

_N_TOP = 2000
_PAD = 2048
_BLK = 256
_NBLK = _PAD // _BLK
_POST = 1000
_OUT_PAD = 1024
_THRESH = 0.7
_MIN_SIZE = 0.001
_IMG = 800.0


def _nms_select_kernel(bt_ref, b_ref, s_ref, out_ref, fs_ref):
    f32 = jnp.float32
    bT = bt_ref[...]
    B4 = b_ref[...]

    x1r = jnp.clip(bT[0:1, :], 0.0, _IMG)
    y1r = jnp.clip(bT[1:2, :], 0.0, _IMG)
    x2r = jnp.clip(bT[2:3, :], 0.0, _IMG)
    y2r = jnp.clip(bT[3:4, :], 0.0, _IMG)
    ws = x2r - x1r
    hs = y2r - y1r
    valid_f = ((ws >= _MIN_SIZE) & (hs >= _MIN_SIZE)).astype(f32)
    area_r = ws * hs

    cols = []
    for q in range(_NBLK):
        Bq = jnp.clip(B4[q * _BLK:(q + 1) * _BLK, :], 0.0, _IMG)
        x1c, y1c = Bq[:, 0:1], Bq[:, 1:2]
        x2c, y2c = Bq[:, 2:3], Bq[:, 3:4]
        cols.append((x1c, y1c, x2c, y2c, (x2c - x1c) * (y2c - y1c)))

    def iou_tile(q, p):
        x1c, y1c, x2c, y2c, ac = cols[q]
        sl = slice(p * _BLK, (p + 1) * _BLK)
        xx1 = jnp.maximum(x1c, x1r[:, sl])
        yy1 = jnp.maximum(y1c, y1r[:, sl])
        xx2 = jnp.minimum(x2c, x2r[:, sl])
        yy2 = jnp.minimum(y2c, y2r[:, sl])
        w = jnp.maximum(xx2 - xx1, 0.0)
        h = jnp.maximum(yy2 - yy1, 0.0)
        inter = w * h
        iou = inter / (ac + area_r[:, sl] - inter + 1e-9)
        return (iou > _THRESH).astype(f32)

    ii = jax.lax.broadcasted_iota(jnp.int32, (_BLK, _BLK), 0)
    jj = jax.lax.broadcasted_iota(jnp.int32, (_BLK, _BLK), 1)
    tri_strict = (ii < jj).astype(f32)
    tri_incl = (ii <= jj).astype(f32)

    keep_blocks = []
    for p in range(_NBLK):
        sup = jnp.zeros((1, _BLK), f32)
        for q in range(p):
            sup = sup + jnp.dot(keep_blocks[q], iou_tile(q, p),
                                preferred_element_type=f32)
        base = valid_f[:, p * _BLK:(p + 1) * _BLK] * (sup == 0.0).astype(f32)
        Mpp = iou_tile(p, p) * tri_strict

        def body(c):
            cur, _ = c
            nxt = base * (jnp.dot(cur, Mpp,
                                  preferred_element_type=f32) == 0.0).astype(f32)
            return (nxt, jnp.any(nxt != cur))

        cur, _ = jax.lax.while_loop(lambda c: c[1], body,
                                    (base, jnp.asarray(True)))
        keep_blocks.append(cur)

    n_kept = jnp.zeros((), f32)
    cs_blocks = []
    for p in range(_NBLK):
        cs_blocks.append(jnp.dot(keep_blocks[p], tri_incl,
                                 preferred_element_type=f32) + n_kept)
        n_kept = n_kept + jnp.sum(keep_blocks[p])
    keep = jnp.concatenate(keep_blocks, axis=1)
    cs = jnp.concatenate(cs_blocks, axis=1)
    pos1 = (jax.lax.broadcasted_iota(jnp.int32, (1, _PAD), 1) + 1).astype(f32)
    rank = jnp.where(keep > 0.0, cs - 1.0, n_kept + (pos1 - cs) - 1.0)

    slot = jax.lax.broadcasted_iota(jnp.int32, (_OUT_PAD, _PAD), 0).astype(f32)
    Pm = slot == rank
    sel = [jnp.max(jnp.where(Pm, rw, -3e38), axis=1, keepdims=True)
           for rw in (x1r, y1r, x2r, y2r, s_ref[...])]

    oi = jax.lax.broadcasted_iota(jnp.int32, (_OUT_PAD, 1), 0).astype(f32)
    out_ref[...] = jnp.concatenate(sel[:4], axis=1)
    fs_ref[...] = jnp.where(oi < n_kept, sel[4], -1e10)


def kernel(boxes, scores):
    top_scores, top_idx = jax.lax.top_k(scores, _N_TOP)
    b = boxes[top_idx]
    pad = _PAD - _N_TOP
    b_p = jnp.concatenate(
        [b, jnp.zeros((pad, 4), jnp.float32)], axis=0)
    s_p = jnp.concatenate(
        [top_scores, jnp.full((pad,), -1e10, jnp.float32)], axis=0)
    bT = b_p.T

    ob, fs = pl.pallas_call(
        _nms_select_kernel,
        out_shape=(
            jax.ShapeDtypeStruct((_OUT_PAD, 4), jnp.float32),
            jax.ShapeDtypeStruct((_OUT_PAD, 1), jnp.float32),
        ),
    )(bT, b_p, s_p[None, :])
    return ob[:_POST, :], fs[:_POST, 0]
